# baseline (speedup 1.0000x reference)
import functools

import jax
import jax.numpy as jnp
from jax import lax
from jax.experimental import pallas as pl
from jax.experimental.pallas import tpu as pltpu
from jax.experimental.pallas import tpu_sc as plsc

B = 1
S = 2048
T = B * S
D = 1024
E = 8
K = 2
H = 2048
LANES = 128
TB = 128
M = 256
NBLK = 23
DUMP = NBLK
P = (NBLK + 1) * M
NC = 2
NS = 16
NW = NC * NS
CH = 16
NEG = -1e30


def _gate_body(x_ref, gw_ref, eidx_ref, gates_ref, rank_ref, stats_ref):
    i = pl.program_id(0)
    nblk = pl.num_programs(0)
    xblk = x_ref[...]
    logits = jnp.dot(xblk, gw_ref[...], preferred_element_type=jnp.float32)
    cols = jax.lax.broadcasted_iota(jnp.int32, (TB, LANES), 1)
    valid = cols < E
    logits = jnp.where(valid, logits, NEG)
    mx = jnp.max(logits, axis=1, keepdims=True)
    ex = jnp.where(valid, jnp.exp(logits - mx), 0.0)
    denom = jnp.sum(ex, axis=1, keepdims=True)
    probs = ex / denom

    v1 = jnp.max(probs, axis=1, keepdims=True)
    i1 = jnp.min(jnp.where(probs == v1, cols, LANES), axis=1, keepdims=True)
    probs2 = jnp.where(cols == i1, -1.0, probs)
    v2 = jnp.max(probs2, axis=1, keepdims=True)
    i2 = jnp.min(jnp.where(probs2 == v2, cols, LANES), axis=1, keepdims=True)

    gd = v1 + v2 + 1e-9
    g1 = v1 / gd
    g2 = v2 / gd
    m1 = (cols == i1).astype(jnp.float32)
    m2 = (cols == i2).astype(jnp.float32)
    lane0 = (cols == 0).astype(jnp.float32)
    lane1 = (cols == 1).astype(jnp.float32)

    @pl.when(i == 0)
    def _():
        stats_ref[...] = jnp.zeros_like(stats_ref)

    run = stats_ref[0:1, :]
    msum = m1 + m2
    rows_i = jax.lax.broadcasted_iota(jnp.int32, (TB, TB), 0)
    cols_i = jax.lax.broadcasted_iota(jnp.int32, (TB, TB), 1)
    tri = (rows_i > cols_i).astype(jnp.float32)
    excl = jnp.dot(tri, msum, preferred_element_type=jnp.float32) + run
    r1 = jnp.sum(excl * m1, axis=1, keepdims=True)
    r2 = jnp.sum(excl * m2, axis=1, keepdims=True)

    eidx_ref[...] = (i1 * (cols == 0) + i2 * (cols == 1)).astype(jnp.int32)
    gates_ref[...] = g1 * lane0 + g2 * lane1
    rank_ref[...] = (r1 * lane0 + r2 * lane1).astype(jnp.int32)

    count_row = jnp.sum(msum, axis=0, keepdims=True)
    m_row = jnp.sum(probs, axis=0, keepdims=True)
    stats_ref[0:1, :] += count_row
    stats_ref[1:2, :] += m_row

    @pl.when(i == nblk - 1)
    def _():
        f = stats_ref[0:1, :] * (1.0 / T)
        m = stats_ref[1:2, :] * (1.0 / T)
        aux = E * jnp.sum(f * m)
        stats_ref[2:3, :] = jnp.full((1, LANES), aux, jnp.float32)


def _vgather16(vec, idx):
    dnums = lax.GatherDimensionNumbers(
        offset_dims=(), collapsed_slice_dims=(0,), start_index_map=(0,))
    return lax.gather(vec, idx[:, None], dnums, (1,),
                      mode=lax.GatherScatterMode.PROMISE_IN_BOUNDS)


def _sc_dispatch_body(x_hbm, e1_hbm, e2_hbm, rk1_hbm, rk2_hbm, segb_hbm,
                      xs_hbm, s0_hbm, s1_hbm,
                      segb_v, e1_v, e2_v, rk1_v, rk2_v, s0_v, s1_v, rows_v,
                      sem):
    wid = lax.axis_index("s") * NC + lax.axis_index("c")
    pltpu.sync_copy(segb_hbm, segb_v)
    tpw = T // NW
    for c in range(tpw // CH):
        base = wid * tpw + c * CH
        pltpu.sync_copy(e1_hbm.at[pl.ds(base, CH)], e1_v)
        pltpu.sync_copy(e2_hbm.at[pl.ds(base, CH)], e2_v)
        pltpu.sync_copy(rk1_hbm.at[pl.ds(base, CH)], rk1_v)
        pltpu.sync_copy(rk2_hbm.at[pl.ds(base, CH)], rk2_v)
        seg_vec = segb_v[...]
        s0_v[...] = _vgather16(seg_vec, e1_v[...]) + rk1_v[...]
        s1_v[...] = _vgather16(seg_vec, e2_v[...]) + rk2_v[...]
        pltpu.sync_copy(s0_v, s0_hbm.at[pl.ds(base, CH)])
        pltpu.sync_copy(s1_v, s1_hbm.at[pl.ds(base, CH)])
        pltpu.sync_copy(x_hbm.at[pl.ds(base, CH)], rows_v)
        pltpu.async_copy(rows_v, xs_hbm.at[s0_v], sem).wait()
        pltpu.async_copy(rows_v, xs_hbm.at[s1_v], sem).wait()


def _ffn_body(be_ref, bp_ref, bv_ref, xs_ref, w1_ref, b1_ref, w2_ref, b2_ref,
              out_ref):
    b = pl.program_id(0)

    @pl.when(bv_ref[b] > 0)
    def _():
        h = jnp.dot(xs_ref[...].astype(jnp.bfloat16), w1_ref[...],
                    preferred_element_type=jnp.float32)
        h = jnp.maximum(h + b1_ref[0], 0.0)
        out_ref[...] = (
            jnp.dot(h.astype(jnp.bfloat16), w2_ref[...],
                    preferred_element_type=jnp.float32)
            + b2_ref[0])


def _sc_gather_body(contrib_hbm, s0_hbm, s1_hbm, r0_hbm, r1_hbm,
                    s0_v, s1_v, rows_v, sem):
    wid = lax.axis_index("s") * NC + lax.axis_index("c")
    tpw = T // NW
    for c in range(tpw // CH):
        base = wid * tpw + c * CH
        pltpu.sync_copy(s0_hbm.at[pl.ds(base, CH)], s0_v)
        pltpu.sync_copy(s1_hbm.at[pl.ds(base, CH)], s1_v)
        pltpu.async_copy(contrib_hbm.at[s0_v], rows_v, sem).wait()
        pltpu.sync_copy(rows_v, r0_hbm.at[pl.ds(base, CH)])
        pltpu.async_copy(contrib_hbm.at[s1_v], rows_v, sem).wait()
        pltpu.sync_copy(rows_v, r1_hbm.at[pl.ds(base, CH)])


def _combine_body(r0_ref, r1_ref, gates_ref, y_ref):
    cols = jax.lax.broadcasted_iota(jnp.int32, (M, LANES), 1)
    g = gates_ref[...]
    g1 = jnp.sum(g * (cols == 0), axis=1, keepdims=True)
    g2 = jnp.sum(g * (cols == 1), axis=1, keepdims=True)
    y_ref[...] = r0_ref[...] * g1 + r1_ref[...] * g2


@jax.jit
def _moe(x, gate_W, W1, b1, W2, b2):
    xt = x.reshape(T, D)
    gwt = jnp.pad(gate_W.T, ((0, 0), (0, LANES - E)))

    eidx, gates, rank, stats = pl.pallas_call(
        _gate_body,
        grid=(T // TB,),
        in_specs=[
            pl.BlockSpec((TB, D), lambda i: (i, 0)),
            pl.BlockSpec((D, LANES), lambda i: (0, 0)),
        ],
        out_specs=[
            pl.BlockSpec((TB, LANES), lambda i: (i, 0)),
            pl.BlockSpec((TB, LANES), lambda i: (i, 0)),
            pl.BlockSpec((TB, LANES), lambda i: (i, 0)),
            pl.BlockSpec((8, LANES), lambda i: (0, 0)),
        ],
        out_shape=[
            jax.ShapeDtypeStruct((T, LANES), jnp.int32),
            jax.ShapeDtypeStruct((T, LANES), jnp.float32),
            jax.ShapeDtypeStruct((T, LANES), jnp.int32),
            jax.ShapeDtypeStruct((8, LANES), jnp.float32),
        ],
    )(xt, gwt)

    counts = stats[0, :E].astype(jnp.int32)
    nblk_e = (counts + M - 1) // M
    csum_b = jnp.cumsum(nblk_e)
    seg_base = (jnp.cumsum(nblk_e * M) - nblk_e * M).astype(jnp.int32)
    blk_starts = csum_b - nblk_e
    total_b = csum_b[E - 1]
    b_ar = jnp.arange(NBLK, dtype=jnp.int32)
    e_of_b = jnp.minimum(
        jnp.searchsorted(csum_b, b_ar, side="right"), E - 1).astype(jnp.int32)
    j_of_b = b_ar - blk_starts[e_of_b]
    valid_b = b_ar < total_b
    blk_expert = jnp.where(valid_b, e_of_b, 0).astype(jnp.int32)
    blk_pos = jnp.where(valid_b, seg_base[e_of_b] // M + j_of_b,
                        DUMP).astype(jnp.int32)
    blk_valid = valid_b.astype(jnp.int32)
    seg_base16 = jnp.pad(seg_base, (0, CH - E))

    e1 = eidx[:, 0]
    e2 = eidx[:, 1]
    rk1 = rank[:, 0]
    rk2 = rank[:, 1]

    mesh = plsc.VectorSubcoreMesh(core_axis_name="c", subcore_axis_name="s")
    xs, s0, s1 = pl.kernel(
        _sc_dispatch_body,
        out_type=[
            jax.ShapeDtypeStruct((P, D), jnp.float32),
            jax.ShapeDtypeStruct((T,), jnp.int32),
            jax.ShapeDtypeStruct((T,), jnp.int32),
        ],
        mesh=mesh,
        scratch_types=[
            pltpu.VMEM((CH,), jnp.int32),
            pltpu.VMEM((CH,), jnp.int32),
            pltpu.VMEM((CH,), jnp.int32),
            pltpu.VMEM((CH,), jnp.int32),
            pltpu.VMEM((CH,), jnp.int32),
            pltpu.VMEM((CH,), jnp.int32),
            pltpu.VMEM((CH,), jnp.int32),
            pltpu.VMEM((CH, D), jnp.float32),
            pltpu.SemaphoreType.DMA,
        ],
    )(xt, e1, e2, rk1, rk2, seg_base16)

    contrib = pl.pallas_call(
        _ffn_body,
        grid_spec=pltpu.PrefetchScalarGridSpec(
            num_scalar_prefetch=3,
            grid=(NBLK,),
            in_specs=[
                pl.BlockSpec((M, D), lambda b, be, bp, bv: (bp[b], 0)),
                pl.BlockSpec((D, H), lambda b, be, bp, bv: (be[b], 0)),
                pl.BlockSpec((1, 1, H), lambda b, be, bp, bv: (be[b], 0, 0)),
                pl.BlockSpec((H, D), lambda b, be, bp, bv: (be[b], 0)),
                pl.BlockSpec((1, 1, D), lambda b, be, bp, bv: (be[b], 0, 0)),
            ],
            out_specs=pl.BlockSpec((M, D), lambda b, be, bp, bv: (bp[b], 0)),
        ),
        out_shape=jax.ShapeDtypeStruct((P, D), jnp.float32),
    )(blk_expert, blk_pos, blk_valid, xs,
      W1.astype(jnp.bfloat16).reshape(E * D, H),
      b1.reshape(E, 1, H),
      W2.astype(jnp.bfloat16).reshape(E * H, D),
      b2.reshape(E, 1, D))

    r0, r1 = pl.kernel(
        _sc_gather_body,
        out_type=[
            jax.ShapeDtypeStruct((T, D), jnp.float32),
            jax.ShapeDtypeStruct((T, D), jnp.float32),
        ],
        mesh=plsc.VectorSubcoreMesh(core_axis_name="c", subcore_axis_name="s"),
        scratch_types=[
            pltpu.VMEM((CH,), jnp.int32),
            pltpu.VMEM((CH,), jnp.int32),
            pltpu.VMEM((CH, D), jnp.float32),
            pltpu.SemaphoreType.DMA,
        ],
    )(contrib, s0, s1)

    y = pl.pallas_call(
        _combine_body,
        grid=(T // M,),
        in_specs=[
            pl.BlockSpec((M, D), lambda i: (i, 0)),
            pl.BlockSpec((M, D), lambda i: (i, 0)),
            pl.BlockSpec((M, LANES), lambda i: (i, 0)),
        ],
        out_specs=pl.BlockSpec((M, D), lambda i: (i, 0)),
        out_shape=jax.ShapeDtypeStruct((T, D), jnp.float32),
    )(r0, r1, gates)

    aux_loss = stats[2, 0]
    tokens_per_expert = stats[0, :E]
    return y.reshape(B, S, D), aux_loss, tokens_per_expert


def kernel(x, gate_W, W1, b1, W2, b2):
    return _moe(x, gate_W, W1, b1, W2, b2)


# speedup vs baseline: 1.3324x; 1.3324x over previous
import functools

import jax
import jax.numpy as jnp
from jax import lax
from jax.experimental import pallas as pl
from jax.experimental.pallas import tpu as pltpu
from jax.experimental.pallas import tpu_sc as plsc

B = 1
S = 2048
T = B * S
D = 1024
E = 8
K = 2
H = 2048
LANES = 128
TB = 256
M = 256
NBLK = 23
DUMP = NBLK
P = (NBLK + 1) * M
NC = 2
NS = 16
NW = NC * NS
TPW = T // NW
L = 16
RSH = 4096
NEG = -1e30


def _gate_body(x_ref, gw_ref, codes_ref, gates_ref, stats_ref):
    i = pl.program_id(0)
    nblk = pl.num_programs(0)
    xblk = x_ref[...]
    logits = jnp.dot(xblk, gw_ref[...], preferred_element_type=jnp.float32)
    cols = jax.lax.broadcasted_iota(jnp.int32, (TB, LANES), 1)
    valid = cols < E
    logits = jnp.where(valid, logits, NEG)
    mx = jnp.max(logits, axis=1, keepdims=True)
    ex = jnp.where(valid, jnp.exp(logits - mx), 0.0)
    denom = jnp.sum(ex, axis=1, keepdims=True)
    probs = ex / denom

    v1 = jnp.max(probs, axis=1, keepdims=True)
    i1 = jnp.min(jnp.where(probs == v1, cols, LANES), axis=1, keepdims=True)
    probs2 = jnp.where(cols == i1, -1.0, probs)
    v2 = jnp.max(probs2, axis=1, keepdims=True)
    i2 = jnp.min(jnp.where(probs2 == v2, cols, LANES), axis=1, keepdims=True)

    gd = v1 + v2 + 1e-9
    g1 = v1 / gd
    g2 = v2 / gd
    m1 = (cols == i1).astype(jnp.float32)
    m2 = (cols == i2).astype(jnp.float32)
    lane0 = (cols == 0).astype(jnp.float32)
    lane1 = (cols == 1).astype(jnp.float32)

    @pl.when(i == 0)
    def _():
        stats_ref[...] = jnp.zeros_like(stats_ref)

    run = stats_ref[0:1, :]
    msum = m1 + m2
    rows_i = jax.lax.broadcasted_iota(jnp.int32, (TB, TB), 0)
    cols_i = jax.lax.broadcasted_iota(jnp.int32, (TB, TB), 1)
    tri = (rows_i > cols_i).astype(jnp.float32)
    excl = jnp.dot(tri, msum, preferred_element_type=jnp.float32) + run
    r1 = jnp.sum(excl * m1, axis=1, keepdims=True).astype(jnp.int32)
    r2 = jnp.sum(excl * m2, axis=1, keepdims=True).astype(jnp.int32)

    code1 = i1 * RSH + r1
    code2 = i2 * RSH + r2
    codes_ref[...] = code1 * (cols == 0) + code2 * (cols == 1)
    gates_ref[...] = g1 * lane0 + g2 * lane1

    count_row = jnp.sum(msum, axis=0, keepdims=True)
    m_row = jnp.sum(probs, axis=0, keepdims=True)
    stats_ref[0:1, :] += count_row
    stats_ref[1:2, :] += m_row

    @pl.when(i == nblk - 1)
    def _():
        f = stats_ref[0:1, :] * (1.0 / T)
        m = stats_ref[1:2, :] * (1.0 / T)
        aux = E * jnp.sum(f * m)
        stats_ref[2:3, :] = jnp.full((1, LANES), aux, jnp.float32)


def _vgather16(vec, idx):
    dnums = lax.GatherDimensionNumbers(
        offset_dims=(), collapsed_slice_dims=(0,), start_index_map=(0,))
    return lax.gather(vec, idx[:, None], dnums, (1,),
                      mode=lax.GatherScatterMode.PROMISE_IN_BOUNDS)


def _sc_dispatch_body(x_hbm, c0_hbm, c1_hbm, segb_hbm,
                      xs_hbm, s0_hbm, s1_hbm,
                      segb_v, c0_v, c1_v, s0_v, s1_v, slab_v,
                      sem_a, sem_b, sem_c):
    wid = lax.axis_index("s") * NC + lax.axis_index("c")
    base = wid * TPW
    slab_cp = pltpu.async_copy(x_hbm.at[pl.ds(base, TPW)], slab_v, sem_a)
    pltpu.sync_copy(segb_hbm, segb_v)
    pltpu.sync_copy(c0_hbm.at[pl.ds(base, TPW)], c0_v)
    pltpu.sync_copy(c1_hbm.at[pl.ds(base, TPW)], c1_v)
    seg_vec = segb_v[...]
    for i in range(TPW // L):
        sl = pl.ds(i * L, L)
        c0 = c0_v[sl]
        c1 = c1_v[sl]
        s0_v[sl] = (_vgather16(seg_vec, lax.shift_right_logical(c0, 12))
                    + (c0 & (RSH - 1)))
        s1_v[sl] = (_vgather16(seg_vec, lax.shift_right_logical(c1, 12))
                    + (c1 & (RSH - 1)))
    pltpu.sync_copy(s0_v, s0_hbm.at[pl.ds(base, TPW)])
    pltpu.sync_copy(s1_v, s1_hbm.at[pl.ds(base, TPW)])
    slab_cp.wait()
    sc0 = pltpu.async_copy(slab_v, xs_hbm.at[s0_v], sem_b)
    sc1 = pltpu.async_copy(slab_v, xs_hbm.at[s1_v], sem_c)
    sc0.wait()
    sc1.wait()


def _ffn_body(be_ref, bp_ref, bv_ref, xs_ref, w1_ref, b1_ref, w2_ref, b2_ref,
              out_ref):
    b = pl.program_id(0)

    @pl.when(bv_ref[b] > 0)
    def _():
        h = jnp.dot(xs_ref[...], w1_ref[...],
                    preferred_element_type=jnp.float32)
        h = jnp.maximum(h + b1_ref[0], 0.0)
        out_ref[...] = (
            jnp.dot(h, w2_ref[...], preferred_element_type=jnp.float32)
            + b2_ref[0])


def _sc_gather_body(contrib_hbm, s0_hbm, s1_hbm, r0_hbm, r1_hbm,
                    s0a_v, s0b_v, s1a_v, s1b_v, buf_a, buf_b,
                    sem_a, sem_b):
    wid = lax.axis_index("s") * NC + lax.axis_index("c")
    base = wid * TPW
    hw = TPW // 2
    pltpu.sync_copy(s0_hbm.at[wid, 0], s0a_v)
    pltpu.sync_copy(s0_hbm.at[wid, 1], s0b_v)
    pltpu.sync_copy(s1_hbm.at[wid, 0], s1a_v)
    pltpu.sync_copy(s1_hbm.at[wid, 1], s1b_v)
    ga = pltpu.async_copy(contrib_hbm.at[s0a_v], buf_a, sem_a)
    gb = pltpu.async_copy(contrib_hbm.at[s0b_v], buf_b, sem_b)
    ga.wait()
    pltpu.sync_copy(buf_a, r0_hbm.at[pl.ds(base, hw)])
    ga2 = pltpu.async_copy(contrib_hbm.at[s1a_v], buf_a, sem_a)
    gb.wait()
    pltpu.sync_copy(buf_b, r0_hbm.at[pl.ds(base + hw, hw)])
    gb2 = pltpu.async_copy(contrib_hbm.at[s1b_v], buf_b, sem_b)
    ga2.wait()
    pltpu.sync_copy(buf_a, r1_hbm.at[pl.ds(base, hw)])
    gb2.wait()
    pltpu.sync_copy(buf_b, r1_hbm.at[pl.ds(base + hw, hw)])


def _combine_body(r0_ref, r1_ref, gates_ref, y_ref):
    cols = jax.lax.broadcasted_iota(jnp.int32, (M, LANES), 1)
    g = gates_ref[...]
    g1 = jnp.sum(g * (cols == 0), axis=1, keepdims=True)
    g2 = jnp.sum(g * (cols == 1), axis=1, keepdims=True)
    y_ref[...] = r0_ref[...] * g1 + r1_ref[...] * g2


@jax.jit
def _moe(x, gate_W, W1, b1, W2, b2):
    xt = x.reshape(T, D)
    gwt = jnp.pad(gate_W.T, ((0, 0), (0, LANES - E)))

    codes, gates, stats = pl.pallas_call(
        _gate_body,
        grid=(T // TB,),
        in_specs=[
            pl.BlockSpec((TB, D), lambda i: (i, 0)),
            pl.BlockSpec((D, LANES), lambda i: (0, 0)),
        ],
        out_specs=[
            pl.BlockSpec((TB, LANES), lambda i: (i, 0)),
            pl.BlockSpec((TB, LANES), lambda i: (i, 0)),
            pl.BlockSpec((8, LANES), lambda i: (0, 0)),
        ],
        out_shape=[
            jax.ShapeDtypeStruct((T, LANES), jnp.int32),
            jax.ShapeDtypeStruct((T, LANES), jnp.float32),
            jax.ShapeDtypeStruct((8, LANES), jnp.float32),
        ],
    )(xt, gwt)

    counts = stats[0, :E].astype(jnp.int32)
    nblk_e = (counts + M - 1) // M
    csum_b = jnp.cumsum(nblk_e)
    seg_base = (jnp.cumsum(nblk_e * M) - nblk_e * M).astype(jnp.int32)
    blk_starts = csum_b - nblk_e
    total_b = csum_b[E - 1]
    b_ar = jnp.arange(NBLK, dtype=jnp.int32)
    e_of_b = jnp.minimum(
        jnp.searchsorted(csum_b, b_ar, side="right"), E - 1).astype(jnp.int32)
    j_of_b = b_ar - blk_starts[e_of_b]
    valid_b = b_ar < total_b
    blk_expert = jnp.where(valid_b, e_of_b, 0).astype(jnp.int32)
    blk_pos = jnp.where(valid_b, seg_base[e_of_b] // M + j_of_b,
                        DUMP).astype(jnp.int32)
    blk_valid = valid_b.astype(jnp.int32)
    seg_base16 = jnp.pad(seg_base, (0, L - E))

    c0 = codes[:, 0]
    c1 = codes[:, 1]

    mesh = plsc.VectorSubcoreMesh(core_axis_name="c", subcore_axis_name="s")
    xs, s0, s1 = pl.kernel(
        _sc_dispatch_body,
        out_type=[
            jax.ShapeDtypeStruct((P, D), jnp.float32),
            jax.ShapeDtypeStruct((T,), jnp.int32),
            jax.ShapeDtypeStruct((T,), jnp.int32),
        ],
        mesh=mesh,
        scratch_types=[
            pltpu.VMEM((L,), jnp.int32),
            pltpu.VMEM((TPW,), jnp.int32),
            pltpu.VMEM((TPW,), jnp.int32),
            pltpu.VMEM((TPW,), jnp.int32),
            pltpu.VMEM((TPW,), jnp.int32),
            pltpu.VMEM((TPW, D), jnp.float32),
            pltpu.SemaphoreType.DMA,
            pltpu.SemaphoreType.DMA,
            pltpu.SemaphoreType.DMA,
        ],
    )(xt, c0, c1, seg_base16)

    contrib = pl.pallas_call(
        _ffn_body,
        grid_spec=pltpu.PrefetchScalarGridSpec(
            num_scalar_prefetch=3,
            grid=(NBLK,),
            in_specs=[
                pl.BlockSpec((M, D), lambda b, be, bp, bv: (bp[b], 0)),
                pl.BlockSpec((D, H), lambda b, be, bp, bv: (be[b], 0)),
                pl.BlockSpec((1, 1, H), lambda b, be, bp, bv: (be[b], 0, 0)),
                pl.BlockSpec((H, D), lambda b, be, bp, bv: (be[b], 0)),
                pl.BlockSpec((1, 1, D), lambda b, be, bp, bv: (be[b], 0, 0)),
            ],
            out_specs=pl.BlockSpec((M, D), lambda b, be, bp, bv: (bp[b], 0)),
        ),
        out_shape=jax.ShapeDtypeStruct((P, D), jnp.float32),
    )(blk_expert, blk_pos, blk_valid, xs, W1.reshape(E * D, H),
      b1.reshape(E, 1, H), W2.reshape(E * H, D), b2.reshape(E, 1, D))

    r0, r1 = pl.kernel(
        _sc_gather_body,
        out_type=[
            jax.ShapeDtypeStruct((T, D), jnp.float32),
            jax.ShapeDtypeStruct((T, D), jnp.float32),
        ],
        mesh=plsc.VectorSubcoreMesh(core_axis_name="c", subcore_axis_name="s"),
        scratch_types=[
            pltpu.VMEM((TPW // 2,), jnp.int32),
            pltpu.VMEM((TPW // 2,), jnp.int32),
            pltpu.VMEM((TPW // 2,), jnp.int32),
            pltpu.VMEM((TPW // 2,), jnp.int32),
            pltpu.VMEM((TPW // 2, D), jnp.float32),
            pltpu.VMEM((TPW // 2, D), jnp.float32),
            pltpu.SemaphoreType.DMA,
            pltpu.SemaphoreType.DMA,
        ],
    )(contrib, s0.reshape(NW, 2, TPW // 2), s1.reshape(NW, 2, TPW // 2))

    y = pl.pallas_call(
        _combine_body,
        grid=(T // M,),
        in_specs=[
            pl.BlockSpec((M, D), lambda i: (i, 0)),
            pl.BlockSpec((M, D), lambda i: (i, 0)),
            pl.BlockSpec((M, LANES), lambda i: (i, 0)),
        ],
        out_specs=pl.BlockSpec((M, D), lambda i: (i, 0)),
        out_shape=jax.ShapeDtypeStruct((T, D), jnp.float32),
    )(r0, r1, gates)

    aux_loss = stats[2, 0]
    tokens_per_expert = stats[0, :E]
    return y.reshape(B, S, D), aux_loss, tokens_per_expert


def kernel(x, gate_W, W1, b1, W2, b2):
    return _moe(x, gate_W, W1, b1, W2, b2)


# speedup vs baseline: 1.3473x; 1.0112x over previous
import functools

import jax
import jax.numpy as jnp
from jax import lax
from jax.experimental import pallas as pl
from jax.experimental.pallas import tpu as pltpu
from jax.experimental.pallas import tpu_sc as plsc

B = 1
S = 2048
T = B * S
D = 1024
E = 8
K = 2
H = 2048
LANES = 128
TB = 256
M = 256
NBLK = 23
DUMP = NBLK
P = (NBLK + 1) * M
NC = 2
NS = 16
NW = NC * NS
TPW = T // NW
L = 16
RSH = 4096
NEG = -1e30


def _gate_body(x_ref, gw_ref, codes_ref, gates_ref, stats_ref, meta_ref):
    i = pl.program_id(0)
    nblk = pl.num_programs(0)
    xblk = x_ref[...]
    logits = jnp.dot(xblk, gw_ref[...], preferred_element_type=jnp.float32)
    cols = jax.lax.broadcasted_iota(jnp.int32, (TB, LANES), 1)
    valid = cols < E
    logits = jnp.where(valid, logits, NEG)
    mx = jnp.max(logits, axis=1, keepdims=True)
    ex = jnp.where(valid, jnp.exp(logits - mx), 0.0)
    denom = jnp.sum(ex, axis=1, keepdims=True)
    probs = ex / denom

    v1 = jnp.max(probs, axis=1, keepdims=True)
    i1 = jnp.min(jnp.where(probs == v1, cols, LANES), axis=1, keepdims=True)
    probs2 = jnp.where(cols == i1, -1.0, probs)
    v2 = jnp.max(probs2, axis=1, keepdims=True)
    i2 = jnp.min(jnp.where(probs2 == v2, cols, LANES), axis=1, keepdims=True)

    gd = v1 + v2 + 1e-9
    g1 = v1 / gd
    g2 = v2 / gd
    m1 = (cols == i1).astype(jnp.float32)
    m2 = (cols == i2).astype(jnp.float32)
    lane0 = (cols == 0).astype(jnp.float32)
    lane1 = (cols == 1).astype(jnp.float32)

    @pl.when(i == 0)
    def _():
        stats_ref[...] = jnp.zeros_like(stats_ref)

    run = stats_ref[0:1, :]
    msum = m1 + m2
    rows_i = jax.lax.broadcasted_iota(jnp.int32, (TB, TB), 0)
    cols_i = jax.lax.broadcasted_iota(jnp.int32, (TB, TB), 1)
    tri = (rows_i > cols_i).astype(jnp.float32)
    excl = jnp.dot(tri, msum, preferred_element_type=jnp.float32) + run
    r1 = jnp.sum(excl * m1, axis=1, keepdims=True).astype(jnp.int32)
    r2 = jnp.sum(excl * m2, axis=1, keepdims=True).astype(jnp.int32)

    code1 = i1 * RSH + r1
    code2 = i2 * RSH + r2
    codes_ref[...] = code1 * (cols == 0) + code2 * (cols == 1)
    gates_ref[...] = g1 * lane0 + g2 * lane1

    count_row = jnp.sum(msum, axis=0, keepdims=True)
    m_row = jnp.sum(probs, axis=0, keepdims=True)
    stats_ref[0:1, :] += count_row
    stats_ref[1:2, :] += m_row

    @pl.when(i == nblk - 1)
    def _():
        f = stats_ref[0:1, :] * (1.0 / T)
        m = stats_ref[1:2, :] * (1.0 / T)
        aux = E * jnp.sum(f * m)
        stats_ref[2:3, :] = jnp.full((1, LANES), aux, jnp.float32)

        counts = stats_ref[0:1, :]
        nblk_f = jnp.floor((counts + (M - 1)) * (1.0 / M))
        li = jax.lax.broadcasted_iota(jnp.int32, (LANES, LANES), 0)
        lj = jax.lax.broadcasted_iota(jnp.int32, (LANES, LANES), 1)
        triu = (li <= lj).astype(jnp.float32)
        csum = jnp.dot(nblk_f, triu, preferred_element_type=jnp.float32)
        seg_excl = (csum - nblk_f) * M
        bst_excl = csum - nblk_f
        total_b = csum[0:1, E - 1:E]
        b_iota = jax.lax.broadcasted_iota(
            jnp.int32, (1, LANES), 1).astype(jnp.float32)
        e_of_b = jnp.zeros((1, LANES), jnp.float32)
        for e in range(E):
            e_of_b = e_of_b + (b_iota >= csum[0:1, e:e + 1]).astype(jnp.float32)
        seg_of_b = jnp.zeros((1, LANES), jnp.float32)
        bst_of_b = jnp.zeros((1, LANES), jnp.float32)
        for e in range(E):
            sel = (e_of_b == e).astype(jnp.float32)
            seg_of_b = seg_of_b + sel * seg_excl[0:1, e:e + 1]
            bst_of_b = bst_of_b + sel * bst_excl[0:1, e:e + 1]
        j_of_b = b_iota - bst_of_b
        validb = b_iota < total_b
        meta_ref[0:1, :] = jnp.where(validb, e_of_b, 0.0).astype(jnp.int32)
        meta_ref[1:2, :] = jnp.where(
            validb, seg_of_b * (1.0 / M) + j_of_b,
            float(DUMP)).astype(jnp.int32)
        meta_ref[2:3, :] = validb.astype(jnp.int32)
        meta_ref[3:4, :] = seg_excl.astype(jnp.int32)


def _vgather16(vec, idx):
    dnums = lax.GatherDimensionNumbers(
        offset_dims=(), collapsed_slice_dims=(0,), start_index_map=(0,))
    return lax.gather(vec, idx[:, None], dnums, (1,),
                      mode=lax.GatherScatterMode.PROMISE_IN_BOUNDS)


def _sc_dispatch_body(x_hbm, c0_hbm, c1_hbm, segb_hbm,
                      xs_hbm, s0_hbm, s1_hbm,
                      segb_v, c0_v, c1_v, s0_v, s1_v, slab_v,
                      sem_a, sem_b, sem_c):
    wid = lax.axis_index("s") * NC + lax.axis_index("c")
    base = wid * TPW
    slab_cp = pltpu.async_copy(x_hbm.at[pl.ds(base, TPW)], slab_v, sem_a)
    pltpu.sync_copy(segb_hbm, segb_v)
    pltpu.sync_copy(c0_hbm.at[pl.ds(base, TPW)], c0_v)
    pltpu.sync_copy(c1_hbm.at[pl.ds(base, TPW)], c1_v)
    seg_vec = segb_v[...]
    for i in range(TPW // L):
        sl = pl.ds(i * L, L)
        c0 = c0_v[sl]
        c1 = c1_v[sl]
        s0_v[sl] = (_vgather16(seg_vec, lax.shift_right_logical(c0, 12))
                    + (c0 & (RSH - 1)))
        s1_v[sl] = (_vgather16(seg_vec, lax.shift_right_logical(c1, 12))
                    + (c1 & (RSH - 1)))
    pltpu.sync_copy(s0_v, s0_hbm.at[pl.ds(base, TPW)])
    pltpu.sync_copy(s1_v, s1_hbm.at[pl.ds(base, TPW)])
    slab_cp.wait()
    sc0 = pltpu.async_copy(slab_v, xs_hbm.at[s0_v], sem_b)
    sc1 = pltpu.async_copy(slab_v, xs_hbm.at[s1_v], sem_c)
    sc0.wait()
    sc1.wait()


def _ffn_body(be_ref, bp_ref, bv_ref, xs_ref, w1_ref, b1_ref, w2_ref, b2_ref,
              out_ref):
    b = pl.program_id(0)

    @pl.when(bv_ref[b] > 0)
    def _():
        h = jnp.dot(xs_ref[...], w1_ref[...],
                    preferred_element_type=jnp.float32)
        h = jnp.maximum(h + b1_ref[0], 0.0)
        out_ref[...] = (
            jnp.dot(h, w2_ref[...], preferred_element_type=jnp.float32)
            + b2_ref[0])


def _sc_gather_body(contrib_hbm, s0_hbm, s1_hbm, r0_hbm, r1_hbm,
                    s0a_v, s0b_v, s1a_v, s1b_v, buf_a, buf_b,
                    sem_a, sem_b):
    wid = lax.axis_index("s") * NC + lax.axis_index("c")
    base = wid * TPW
    hw = TPW // 2
    pltpu.sync_copy(s0_hbm.at[wid, 0], s0a_v)
    pltpu.sync_copy(s0_hbm.at[wid, 1], s0b_v)
    pltpu.sync_copy(s1_hbm.at[wid, 0], s1a_v)
    pltpu.sync_copy(s1_hbm.at[wid, 1], s1b_v)
    ga = pltpu.async_copy(contrib_hbm.at[s0a_v], buf_a, sem_a)
    gb = pltpu.async_copy(contrib_hbm.at[s0b_v], buf_b, sem_b)
    ga.wait()
    pltpu.sync_copy(buf_a, r0_hbm.at[pl.ds(base, hw)])
    ga2 = pltpu.async_copy(contrib_hbm.at[s1a_v], buf_a, sem_a)
    gb.wait()
    pltpu.sync_copy(buf_b, r0_hbm.at[pl.ds(base + hw, hw)])
    gb2 = pltpu.async_copy(contrib_hbm.at[s1b_v], buf_b, sem_b)
    ga2.wait()
    pltpu.sync_copy(buf_a, r1_hbm.at[pl.ds(base, hw)])
    gb2.wait()
    pltpu.sync_copy(buf_b, r1_hbm.at[pl.ds(base + hw, hw)])


def _combine_body(r0_ref, r1_ref, gates_ref, y_ref):
    cols = jax.lax.broadcasted_iota(jnp.int32, (M, LANES), 1)
    g = gates_ref[...]
    g1 = jnp.sum(g * (cols == 0), axis=1, keepdims=True)
    g2 = jnp.sum(g * (cols == 1), axis=1, keepdims=True)
    y_ref[...] = r0_ref[...] * g1 + r1_ref[...] * g2


@jax.jit
def _moe(x, gate_W, W1, b1, W2, b2):
    xt = x.reshape(T, D)
    gwt = jnp.pad(gate_W.T, ((0, 0), (0, LANES - E)))

    codes, gates, stats, meta = pl.pallas_call(
        _gate_body,
        grid=(T // TB,),
        in_specs=[
            pl.BlockSpec((TB, D), lambda i: (i, 0)),
            pl.BlockSpec((D, LANES), lambda i: (0, 0)),
        ],
        out_specs=[
            pl.BlockSpec((TB, LANES), lambda i: (i, 0)),
            pl.BlockSpec((TB, LANES), lambda i: (i, 0)),
            pl.BlockSpec((8, LANES), lambda i: (0, 0)),
            pl.BlockSpec((8, LANES), lambda i: (0, 0)),
        ],
        out_shape=[
            jax.ShapeDtypeStruct((T, LANES), jnp.int32),
            jax.ShapeDtypeStruct((T, LANES), jnp.float32),
            jax.ShapeDtypeStruct((8, LANES), jnp.float32),
            jax.ShapeDtypeStruct((8, LANES), jnp.int32),
        ],
    )(xt, gwt)

    blk_expert = meta[0, :NBLK]
    blk_pos = meta[1, :NBLK]
    blk_valid = meta[2, :NBLK]
    seg_base16 = meta[3, :L]
    c0 = codes[:, 0]
    c1 = codes[:, 1]

    mesh = plsc.VectorSubcoreMesh(core_axis_name="c", subcore_axis_name="s")
    xs, s0, s1 = pl.kernel(
        _sc_dispatch_body,
        out_type=[
            jax.ShapeDtypeStruct((P, D), jnp.float32),
            jax.ShapeDtypeStruct((T,), jnp.int32),
            jax.ShapeDtypeStruct((T,), jnp.int32),
        ],
        mesh=mesh,
        scratch_types=[
            pltpu.VMEM((L,), jnp.int32),
            pltpu.VMEM((TPW,), jnp.int32),
            pltpu.VMEM((TPW,), jnp.int32),
            pltpu.VMEM((TPW,), jnp.int32),
            pltpu.VMEM((TPW,), jnp.int32),
            pltpu.VMEM((TPW, D), jnp.float32),
            pltpu.SemaphoreType.DMA,
            pltpu.SemaphoreType.DMA,
            pltpu.SemaphoreType.DMA,
        ],
    )(xt, c0, c1, seg_base16)

    contrib = pl.pallas_call(
        _ffn_body,
        grid_spec=pltpu.PrefetchScalarGridSpec(
            num_scalar_prefetch=3,
            grid=(NBLK,),
            in_specs=[
                pl.BlockSpec((M, D), lambda b, be, bp, bv: (bp[b], 0)),
                pl.BlockSpec((D, H), lambda b, be, bp, bv: (be[b], 0)),
                pl.BlockSpec((1, 1, H), lambda b, be, bp, bv: (be[b], 0, 0)),
                pl.BlockSpec((H, D), lambda b, be, bp, bv: (be[b], 0)),
                pl.BlockSpec((1, 1, D), lambda b, be, bp, bv: (be[b], 0, 0)),
            ],
            out_specs=pl.BlockSpec((M, D), lambda b, be, bp, bv: (bp[b], 0)),
        ),
        out_shape=jax.ShapeDtypeStruct((P, D), jnp.float32),
    )(blk_expert, blk_pos, blk_valid, xs, W1.reshape(E * D, H),
      b1.reshape(E, 1, H), W2.reshape(E * H, D), b2.reshape(E, 1, D))

    r0, r1 = pl.kernel(
        _sc_gather_body,
        out_type=[
            jax.ShapeDtypeStruct((T, D), jnp.float32),
            jax.ShapeDtypeStruct((T, D), jnp.float32),
        ],
        mesh=plsc.VectorSubcoreMesh(core_axis_name="c", subcore_axis_name="s"),
        scratch_types=[
            pltpu.VMEM((TPW // 2,), jnp.int32),
            pltpu.VMEM((TPW // 2,), jnp.int32),
            pltpu.VMEM((TPW // 2,), jnp.int32),
            pltpu.VMEM((TPW // 2,), jnp.int32),
            pltpu.VMEM((TPW // 2, D), jnp.float32),
            pltpu.VMEM((TPW // 2, D), jnp.float32),
            pltpu.SemaphoreType.DMA,
            pltpu.SemaphoreType.DMA,
        ],
    )(contrib, s0.reshape(NW, 2, TPW // 2), s1.reshape(NW, 2, TPW // 2))

    y = pl.pallas_call(
        _combine_body,
        grid=(T // M,),
        in_specs=[
            pl.BlockSpec((M, D), lambda i: (i, 0)),
            pl.BlockSpec((M, D), lambda i: (i, 0)),
            pl.BlockSpec((M, LANES), lambda i: (i, 0)),
        ],
        out_specs=pl.BlockSpec((M, D), lambda i: (i, 0)),
        out_shape=jax.ShapeDtypeStruct((T, D), jnp.float32),
    )(r0, r1, gates)

    aux_loss = stats[2, 0]
    tokens_per_expert = stats[0, :E]
    return y.reshape(B, S, D), aux_loss, tokens_per_expert


def kernel(x, gate_W, W1, b1, W2, b2):
    return _moe(x, gate_W, W1, b1, W2, b2)
